# baseline (speedup 1.0000x reference)
import jax
import jax.numpy as jnp
from jax.experimental import pallas as pl
from jax.experimental.pallas import tpu as pltpu

_BM = 512
_NBUF = 4
_NOUT = 2


def _matmul_prelu_kernel(
    w_ref, adj_hbm, seq_hbm, out_hbm, bufs, seq_vmem, obufs, sems, seq_sem, osems
):
    n = adj_hbm.shape[0]
    nchunk = n // _BM

    def adj_copy(chunk):
        buf = chunk % _NBUF
        return pltpu.make_async_copy(
            adj_hbm.at[pl.ds(chunk * _BM, _BM), :], bufs.at[buf], sems.at[buf]
        )

    def out_copy(chunk):
        ob = chunk % _NOUT
        return pltpu.make_async_copy(
            obufs.at[ob], out_hbm.at[pl.ds(chunk * _BM, _BM), :], osems.at[ob]
        )

    seq_cp = pltpu.make_async_copy(seq_hbm, seq_vmem, seq_sem)
    seq_cp.start()
    for j in range(min(_NBUF, nchunk)):
        adj_copy(j).start()
    seq_cp.wait()

    w = w_ref[0, 0]
    for i in range(nchunk):
        adj_copy(i).wait()
        y = jnp.dot(
            bufs[i % _NBUF], seq_vmem[...], preferred_element_type=jnp.float32
        )
        if i >= _NOUT:
            out_copy(i - _NOUT).wait()
        obufs[i % _NOUT] = jnp.where(y >= 0, y, w * y)
        out_copy(i).start()
        nxt = i + _NBUF
        if nxt < nchunk:
            adj_copy(nxt).start()

    for i in range(max(0, nchunk - _NOUT), nchunk):
        out_copy(i).wait()


def kernel(seq, adj_avg, prelu_weight):
    n, d = seq.shape
    w2d = prelu_weight.reshape(1, 1)
    return pl.pallas_call(
        _matmul_prelu_kernel,
        in_specs=[
            pl.BlockSpec(memory_space=pltpu.SMEM),
            pl.BlockSpec(memory_space=pltpu.MemorySpace.HBM),
            pl.BlockSpec(memory_space=pltpu.MemorySpace.HBM),
        ],
        out_specs=pl.BlockSpec(memory_space=pltpu.MemorySpace.HBM),
        out_shape=jax.ShapeDtypeStruct((n, d), jnp.float32),
        scratch_shapes=[
            pltpu.VMEM((_NBUF, _BM, n), jnp.float32),
            pltpu.VMEM((n, d), jnp.float32),
            pltpu.VMEM((_NOUT, _BM, d), jnp.float32),
            pltpu.SemaphoreType.DMA((_NBUF,)),
            pltpu.SemaphoreType.DMA,
            pltpu.SemaphoreType.DMA((_NOUT,)),
        ],
    )(w2d, adj_avg, seq)


# speedup vs baseline: 1.0898x; 1.0898x over previous
import jax
import jax.numpy as jnp
from jax.experimental import pallas as pl

_BM = 512


def _matmul_prelu_kernel(w_ref, adj_a, adj_b, seq_ref, out_ref):
    w = w_ref[0, 0]
    ya = jnp.dot(adj_a[...], seq_ref[...], preferred_element_type=jnp.float32)
    out_ref[: _BM, :] = jnp.where(ya >= 0, ya, w * ya)
    yb = jnp.dot(adj_b[...], seq_ref[...], preferred_element_type=jnp.float32)
    out_ref[_BM :, :] = jnp.where(yb >= 0, yb, w * yb)


def kernel(seq, adj_avg, prelu_weight):
    n, d = seq.shape
    w2d = prelu_weight.reshape(1, 1)
    grid = (n // (2 * _BM),)
    return pl.pallas_call(
        _matmul_prelu_kernel,
        grid=grid,
        in_specs=[
            pl.BlockSpec((1, 1), lambda i: (0, 0)),
            pl.BlockSpec((_BM, n), lambda i: (2 * i, 0)),
            pl.BlockSpec((_BM, n), lambda i: (2 * i + 1, 0)),
            pl.BlockSpec((n, d), lambda i: (0, 0)),
        ],
        out_specs=pl.BlockSpec((2 * _BM, d), lambda i: (i, 0)),
        out_shape=jax.ShapeDtypeStruct((n, d), jnp.float32),
    )(w2d, adj_avg, adj_avg, seq)


# speedup vs baseline: 1.1684x; 1.0721x over previous
import jax
import jax.numpy as jnp
from jax.experimental import pallas as pl

_BM = 512


def _matmul_prelu_kernel(w_ref, adj_ref, seq_ref, out_ref):
    y = jnp.dot(adj_ref[...], seq_ref[...], preferred_element_type=jnp.float32)
    w = w_ref[0, 0]
    out_ref[...] = jnp.where(y >= 0, y, w * y)


def kernel(seq, adj_avg, prelu_weight):
    n, d = seq.shape
    w2d = prelu_weight.reshape(1, 1)
    grid = (n // _BM,)
    return pl.pallas_call(
        _matmul_prelu_kernel,
        grid=grid,
        in_specs=[
            pl.BlockSpec((1, 1), lambda i: (0, 0)),
            pl.BlockSpec((_BM, n), lambda i: (i, 0)),
            pl.BlockSpec((n, d), lambda i: (0, 0)),
        ],
        out_specs=pl.BlockSpec((_BM, d), lambda i: (i, 0)),
        out_shape=jax.ShapeDtypeStruct((n, d), jnp.float32),
    )(w2d, adj_avg, seq)
